# Initial kernel scaffold; baseline (speedup 1.0000x reference)
import functools
import math

import jax
import jax.numpy as jnp
from jax import lax
from jax.experimental import pallas as pl
from jax.experimental.pallas import tpu as pltpu
from jax.experimental.pallas import tpu_sc as plsc

D = 32
SCALE = math.sqrt(D)
NC, NS = 2, 16
NW = NC * NS
L = 16
CH = 1024
GW = 128
NG = CH // GW


def _make_kernel(B: int):
    rows_per_w = B // NW
    nchunk = rows_per_w // CH
    xrow0_per_w = rows_per_w // GW

    @functools.partial(
        pl.kernel,
        out_type=jax.ShapeDtypeStruct((B, D), jnp.float32),
        mesh=plsc.VectorSubcoreMesh(core_axis_name="c", subcore_axis_name="s"),
        scratch_types=[
            pltpu.VMEM((NG, GW), jnp.int32),
            pltpu.VMEM((CH, D), jnp.float32),
            pltpu.SemaphoreType.DMA,
        ],
    )
    def run(x_ref, t_ref, o_ref, idx_v, rows_v, gsem):
        wid = lax.axis_index("s") * NC + lax.axis_index("c")
        xrow0 = wid * xrow0_per_w

        def chunk(c, carry):
            rb = xrow0 + c * NG
            pltpu.sync_copy(x_ref.at[pl.ds(rb, NG)], idx_v)
            descs = [
                pltpu.async_copy(
                    t_ref.at[idx_v.at[j]],
                    rows_v.at[pl.ds(j * GW, GW)],
                    gsem,
                )
                for j in range(NG)
            ]
            for d in descs:
                d.wait()

            def scale(i, c2):
                rows_v[i, pl.ds(0, L)] = rows_v[i, pl.ds(0, L)] * SCALE
                rows_v[i, pl.ds(L, L)] = rows_v[i, pl.ds(L, L)] * SCALE
                return c2

            lax.fori_loop(0, CH, scale, 0)
            pltpu.sync_copy(rows_v, o_ref.at[pl.ds(rb * GW, CH)])
            return carry

        lax.fori_loop(0, nchunk, chunk, 0)

    return run


def kernel(x, table):
    B = x.shape[0] * x.shape[1]
    xf = x.reshape(B // GW, GW).astype(jnp.int32)
    out = _make_kernel(B)(xf, table)
    return out.reshape(x.shape + (D,))


# speedup vs baseline: 1.2921x; 1.2921x over previous
import functools
import math

import jax
import jax.numpy as jnp
from jax import lax
from jax.experimental import pallas as pl
from jax.experimental.pallas import tpu as pltpu
from jax.experimental.pallas import tpu_sc as plsc

D = 32
SCALE = math.sqrt(D)
NC, NS = 2, 16
NW = NC * NS
L = 16
CH = 1024
GW = 128
NG = CH // GW


def _make_kernel(B: int):
    rows_per_w = B // NW
    nchunk = rows_per_w // CH
    xrow0_per_w = rows_per_w // GW

    @functools.partial(
        pl.kernel,
        out_type=jax.ShapeDtypeStruct((B, D), jnp.float32),
        mesh=plsc.VectorSubcoreMesh(core_axis_name="c", subcore_axis_name="s"),
        scratch_types=[
            pltpu.VMEM((NG, GW), jnp.int32),
            pltpu.VMEM((CH, D), jnp.float32),
            pltpu.SemaphoreType.DMA,
        ],
        compiler_params=pltpu.CompilerParams(use_tc_tiling_on_sc=False),
    )
    def run(x_ref, t_ref, o_ref, idx_v, rows_v, gsem):
        wid = lax.axis_index("s") * NC + lax.axis_index("c")
        xrow0 = wid * xrow0_per_w

        def chunk(c, carry):
            rb = xrow0 + c * NG
            pltpu.sync_copy(x_ref.at[pl.ds(rb, NG)], idx_v)
            descs = [
                pltpu.async_copy(
                    t_ref.at[idx_v.at[j]],
                    rows_v.at[pl.ds(j * GW, GW)],
                    gsem,
                )
                for j in range(NG)
            ]
            for d in descs:
                d.wait()

            def scale(i, c2):
                rows_v[i, pl.ds(0, L)] = rows_v[i, pl.ds(0, L)] * SCALE
                rows_v[i, pl.ds(L, L)] = rows_v[i, pl.ds(L, L)] * SCALE
                return c2

            lax.fori_loop(0, CH, scale, 0)
            pltpu.sync_copy(rows_v, o_ref.at[pl.ds(rb * GW, CH)])
            return carry

        lax.fori_loop(0, nchunk, chunk, 0)

    return run


def kernel(x, table):
    B = x.shape[0] * x.shape[1]
    xf = x.reshape(B // GW, GW).astype(jnp.int32)
    out = _make_kernel(B)(xf, table)
    return out.reshape(x.shape + (D,))


# speedup vs baseline: 1.4786x; 1.1443x over previous
import functools
import math

import jax
import jax.numpy as jnp
from jax import lax
from jax.experimental import pallas as pl
from jax.experimental.pallas import tpu as pltpu
from jax.experimental.pallas import tpu_sc as plsc

D = 32
SCALE = math.sqrt(D)
NC, NS = 2, 16
NW = NC * NS
L = 16
GW = 128
NG = 5
CH = NG * GW
NBUF = 4


def _make_kernel(B: int):
    rows_per_w = B // NW
    nchunk = rows_per_w // CH
    nxrow = rows_per_w // GW
    npair = nchunk // NBUF

    @functools.partial(
        pl.kernel,
        out_type=jax.ShapeDtypeStruct((B, D), jnp.float32),
        mesh=plsc.VectorSubcoreMesh(core_axis_name="c", subcore_axis_name="s"),
        scratch_types=[
            pltpu.VMEM((nxrow, GW), jnp.int32),
            [pltpu.VMEM((CH, D), jnp.float32) for _ in range(NBUF)],
            [pltpu.SemaphoreType.DMA for _ in range(NBUF)],
            [pltpu.SemaphoreType.DMA for _ in range(NBUF)],
        ],
        compiler_params=pltpu.CompilerParams(use_tc_tiling_on_sc=False),
    )
    def run(x_ref, t_ref, o_ref, idx_all, rows, gsem, osem):
        wid = lax.axis_index("s") * NC + lax.axis_index("c")
        obase = wid * rows_per_w

        pltpu.sync_copy(x_ref.at[pl.ds(wid * nxrow, nxrow)], idx_all)

        def fire(c, k):
            rb = c * NG
            for j in range(NG):
                pltpu.async_copy(
                    t_ref.at[idx_all.at[rb + j]],
                    rows[k].at[pl.ds(j * GW, GW)],
                    gsem[k],
                )

        def wait_gathers(k):
            pltpu.make_async_copy(o_ref.at[pl.ds(0, CH)], rows[k], gsem[k]).wait()

        def wait_store(k):
            pltpu.make_async_copy(rows[k], o_ref.at[pl.ds(0, CH)], osem[k]).wait()

        for k in range(NBUF - 1):
            fire(k, k)

        def pair(t, carry):
            for k in range(NBUF):
                c = NBUF * t + k
                wait_gathers(k)

                rv = rows[k]

                @plsc.parallel_loop(0, CH, step=1, unroll=8)
                def scale(i):
                    rv[i, pl.ds(0, L)] = rv[i, pl.ds(0, L)] * SCALE
                    rv[i, pl.ds(L, L)] = rv[i, pl.ds(L, L)] * SCALE

                pltpu.async_copy(
                    rows[k], o_ref.at[pl.ds(obase + c * CH, CH)], osem[k]
                )

                kb = (k + NBUF - 1) % NBUF
                if k == 0:
                    @pl.when(t > 0)
                    def _():
                        wait_store(kb)
                        fire(c + NBUF - 1, kb)

                    @pl.when(t == 0)
                    def _():
                        fire(c + NBUF - 1, kb)
                else:
                    @pl.when(c + NBUF - 1 < nchunk)
                    def _():
                        wait_store(kb)
                        fire(c + NBUF - 1, kb)
            return carry

        lax.fori_loop(0, npair, pair, 0)
        for k in range(NBUF):
            wait_store(k)

    return run


def kernel(x, table):
    B = x.shape[0] * x.shape[1]
    xf = x.reshape(B // GW, GW).astype(jnp.int32)
    out = _make_kernel(B)(xf, table)
    return out.reshape(x.shape + (D,))
